# baseline (speedup 1.0000x reference)
import functools

import jax
import jax.numpy as jnp
from jax import lax
from jax.experimental import pallas as pl
from jax.experimental.pallas import tpu as pltpu
from jax.experimental.pallas import tpu_sc as plsc

N = 8192
L = 16
NC = 2
NS = 16
NW = NC * NS
QPW = N // NW
QG = 4
BIG = jnp.float32(1e30)

_TAKE_DNUMS = lax.GatherDimensionNumbers(
    offset_dims=(), collapsed_slice_dims=(0,), start_index_map=(0,))


def _take16(vec, idx):
    return lax.gather(vec, idx[:, None], _TAKE_DNUMS, slice_sizes=(1,),
                      mode=lax.GatherScatterMode.PROMISE_IN_BOUNDS)


def _bcast_lane(vec, lane):
    return _take16(vec, jnp.full((L,), lane, jnp.int32))


def _lane_argmin(val, idx, perms):
    for p in perms:
        vo = _take16(val, p)
        io = _take16(idx, p)
        better = (vo < val) | ((vo == val) & (io < idx))
        val = jnp.where(better, vo, val)
        idx = jnp.where(better, io, idx)
    return val, idx


def _phase1_body(hqx_h, hqy_h, hqz_h, hcx_h, hcy_h, hcz_h,
                 qx_h, qy_h, qz_h, cx_h, cy_h, cz_h,
                 sq1_h, sq2_h, ids_h,
                 rowd2_h, colval_h, cold2_h,
                 hcx, hcy, hcz, cx, cy, cz, sq2_v, ids_v,
                 colval_v, colidx_v, cold2_v,
                 hqx, hqy, hqz, qx, qy, qz, sq1_v, rowd2_v):
    wid = lax.axis_index("s") * NC + lax.axis_index("c")
    base = wid * QPW

    pltpu.sync_copy(hcx_h, hcx)
    pltpu.sync_copy(hcy_h, hcy)
    pltpu.sync_copy(hcz_h, hcz)
    pltpu.sync_copy(cx_h, cx)
    pltpu.sync_copy(cy_h, cy)
    pltpu.sync_copy(cz_h, cz)
    pltpu.sync_copy(sq2_h, sq2_v)
    pltpu.sync_copy(ids_h, ids_v)
    pltpu.sync_copy(hqx_h.at[pl.ds(base, QPW)], hqx)
    pltpu.sync_copy(hqy_h.at[pl.ds(base, QPW)], hqy)
    pltpu.sync_copy(hqz_h.at[pl.ds(base, QPW)], hqz)
    pltpu.sync_copy(qx_h.at[pl.ds(base, QPW)], qx)
    pltpu.sync_copy(qy_h.at[pl.ds(base, QPW)], qy)
    pltpu.sync_copy(qz_h.at[pl.ds(base, QPW)], qz)
    pltpu.sync_copy(sq1_h.at[pl.ds(base, QPW)], sq1_v)

    iota = lax.iota(jnp.int32, L)
    perms = [jnp.bitwise_and(iota + sh, L - 1) for sh in (8, 4, 2, 1)]
    big_vec = jnp.full((L,), BIG, jnp.float32)
    zero_i = jnp.zeros((L,), jnp.int32)

    def init_body(i, _):
        s = pl.ds(i * L, L)
        colval_v[s] = big_vec
        colidx_v[s] = zero_i
        return 0

    lax.fori_loop(0, N // L, init_body, 0)

    def group_body(g, _):
        qb = g * L
        hqxv = hqx[pl.ds(qb, L)]
        hqyv = hqy[pl.ds(qb, L)]
        hqzv = hqz[pl.ds(qb, L)]
        sq1v = sq1_v[pl.ds(qb, L)]
        res_idx = zero_i
        for sub in range(L // QG):
            lanes = [sub * QG + t for t in range(QG)]
            bqx = [_bcast_lane(hqxv, l) for l in lanes]
            bqy = [_bcast_lane(hqyv, l) for l in lanes]
            bqz = [_bcast_lane(hqzv, l) for l in lanes]
            bs1 = [_bcast_lane(sq1v, l) for l in lanes]
            biv = [jnp.full((L,), qb + l, jnp.int32) for l in lanes]

            def jbody(jb, carry):
                rvals = list(carry[:QG])
                ridxs = list(carry[QG:])
                s = pl.ds(jb * L, L)
                xv = hcx[s]
                yv = hcy[s]
                zv = hcz[s]
                s2 = sq2_v[s]
                jvec = ids_v[s]
                cval = colval_v[s]
                cidx = colidx_v[s]
                for t in range(QG):
                    m = bs1[t] + s2
                    m = m - bqx[t] * xv
                    m = m - bqy[t] * yv
                    m = m - bqz[t] * zv
                    lt = m < rvals[t]
                    rvals[t] = jnp.where(lt, m, rvals[t])
                    ridxs[t] = jnp.where(lt, jvec, ridxs[t])
                    clt = m < cval
                    cval = jnp.where(clt, m, cval)
                    cidx = jnp.where(clt, biv[t], cidx)
                colval_v[s] = cval
                colidx_v[s] = cidx
                return tuple(rvals) + tuple(ridxs)

            carry0 = tuple(big_vec for _ in range(QG)) + \
                tuple(zero_i for _ in range(QG))
            out = lax.fori_loop(0, N // L, jbody, carry0)
            for t in range(QG):
                _, idx_r = _lane_argmin(out[t], out[QG + t], perms)
                res_idx = jnp.where(iota == lanes[t], idx_r, res_idx)
        gx = plsc.load_gather(cx, [res_idx])
        gy = plsc.load_gather(cy, [res_idx])
        gz = plsc.load_gather(cz, [res_idx])
        dx = qx[pl.ds(qb, L)] - gx
        dy = qy[pl.ds(qb, L)] - gy
        dz = qz[pl.ds(qb, L)] - gz
        rowd2_v[pl.ds(qb, L)] = dx * dx + dy * dy + dz * dz
        return 0

    lax.fori_loop(0, QPW // L, group_body, 0)

    def col_body(jv, _):
        s = pl.ds(jv * L, L)
        lidx = colidx_v[s]
        gx = plsc.load_gather(qx, [lidx])
        gy = plsc.load_gather(qy, [lidx])
        gz = plsc.load_gather(qz, [lidx])
        dx = cx[s] - gx
        dy = cy[s] - gy
        dz = cz[s] - gz
        cold2_v[s] = dx * dx + dy * dy + dz * dz
        return 0

    lax.fori_loop(0, N // L, col_body, 0)

    pltpu.sync_copy(rowd2_v, rowd2_h.at[pl.ds(base, QPW)])
    pltpu.sync_copy(colval_v, colval_h.at[wid])
    pltpu.sync_copy(cold2_v, cold2_h.at[wid])


_phase1 = functools.partial(
    pl.kernel,
    out_type=[
        jax.ShapeDtypeStruct((N,), jnp.float32),
        jax.ShapeDtypeStruct((NW, N), jnp.float32),
        jax.ShapeDtypeStruct((NW, N), jnp.float32),
    ],
    mesh=plsc.VectorSubcoreMesh(core_axis_name="c", subcore_axis_name="s"),
    compiler_params=pltpu.CompilerParams(needs_layout_passes=False),
    scratch_types=[
        pltpu.VMEM((N,), jnp.float32),
        pltpu.VMEM((N,), jnp.float32),
        pltpu.VMEM((N,), jnp.float32),
        pltpu.VMEM((N,), jnp.float32),
        pltpu.VMEM((N,), jnp.float32),
        pltpu.VMEM((N,), jnp.float32),
        pltpu.VMEM((N,), jnp.float32),
        pltpu.VMEM((N,), jnp.int32),
        pltpu.VMEM((N,), jnp.float32),
        pltpu.VMEM((N,), jnp.int32),
        pltpu.VMEM((N,), jnp.float32),
        pltpu.VMEM((QPW,), jnp.float32),
        pltpu.VMEM((QPW,), jnp.float32),
        pltpu.VMEM((QPW,), jnp.float32),
        pltpu.VMEM((QPW,), jnp.float32),
        pltpu.VMEM((QPW,), jnp.float32),
        pltpu.VMEM((QPW,), jnp.float32),
        pltpu.VMEM((QPW,), jnp.float32),
        pltpu.VMEM((QPW,), jnp.float32),
    ],
)(_phase1_body)


_ROWS = N // 128


def _epilogue_body(row_ref, colval_ref, cold2_ref, out_ref):
    val = colval_ref[pl.ds(0, _ROWS), :]
    d2 = cold2_ref[pl.ds(0, _ROWS), :]
    for w in range(1, NW):
        v2 = colval_ref[pl.ds(w * _ROWS, _ROWS), :]
        c2 = cold2_ref[pl.ds(w * _ROWS, _ROWS), :]
        lt = v2 < val
        val = jnp.where(lt, v2, val)
        d2 = jnp.where(lt, c2, d2)
    s1 = jnp.sum(jnp.sqrt(row_ref[...]))
    s2 = jnp.sum(jnp.sqrt(d2))
    out_ref[0, 0] = (s1 + s2) * jnp.float32(1.0 / N)


_epilogue = pl.pallas_call(
    _epilogue_body,
    out_shape=jax.ShapeDtypeStruct((1, 1), jnp.float32),
    in_specs=[
        pl.BlockSpec(memory_space=pltpu.VMEM),
        pl.BlockSpec(memory_space=pltpu.VMEM),
        pl.BlockSpec(memory_space=pltpu.VMEM),
    ],
    out_specs=pl.BlockSpec(memory_space=pltpu.SMEM),
)


def kernel(y1, y2, b1, b2):
    del b1, b2
    h1 = lax.optimization_barrier(y1.astype(jnp.bfloat16)).astype(jnp.float32)
    h2 = lax.optimization_barrier(y2.astype(jnp.bfloat16)).astype(jnp.float32)
    sq1 = jnp.sum(y1 * y1, axis=1)
    sq2 = jnp.sum(y2 * y2, axis=1)
    ids = jnp.arange(N, dtype=jnp.int32)
    h2d = h2 + h2
    rowd2, colval, cold2 = _phase1(
        h1[:, 0], h1[:, 1], h1[:, 2], h2d[:, 0], h2d[:, 1], h2d[:, 2],
        y1[:, 0], y1[:, 1], y1[:, 2], y2[:, 0], y2[:, 1], y2[:, 2],
        sq1, sq2, ids)
    out = _epilogue(rowd2.reshape(_ROWS, 128),
                    colval.reshape(NW * _ROWS, 128),
                    cold2.reshape(NW * _ROWS, 128))
    return out[0, 0]


# speedup vs baseline: 1.0259x; 1.0259x over previous
import functools

import jax
import jax.numpy as jnp
from jax import lax
from jax.experimental import pallas as pl
from jax.experimental.pallas import tpu as pltpu
from jax.experimental.pallas import tpu_sc as plsc

N = 8192
L = 16
NC = 2
NS = 16
NW = NC * NS
QPW = N // NW
QG = 4
BIG = jnp.float32(1e30)

_TAKE_DNUMS = lax.GatherDimensionNumbers(
    offset_dims=(), collapsed_slice_dims=(0,), start_index_map=(0,))


def _take16(vec, idx):
    return lax.gather(vec, idx[:, None], _TAKE_DNUMS, slice_sizes=(1,),
                      mode=lax.GatherScatterMode.PROMISE_IN_BOUNDS)


def _bcast_lane(vec, lane):
    return _take16(vec, jnp.full((L,), lane, jnp.int32))


def _lane_argmin(val, idx, perms):
    for p in perms:
        vo = _take16(val, p)
        io = _take16(idx, p)
        better = (vo < val) | ((vo == val) & (io < idx))
        val = jnp.where(better, vo, val)
        idx = jnp.where(better, io, idx)
    return val, idx


def _phase1_body(hqx_h, hqy_h, hqz_h, hcx_h, hcy_h, hcz_h,
                 qx_h, qy_h, qz_h, cx_h, cy_h, cz_h,
                 sq1_h, sq2_h, ids_h,
                 rowd2_h, colval_h, cold2_h,
                 hcx, hcy, hcz, cx, cy, cz, sq2_v, ids_v,
                 colval_v, colidx_v, cold2_v,
                 hqx, hqy, hqz, qx, qy, qz, sq1_v, rowd2_v):
    wid = lax.axis_index("s") * NC + lax.axis_index("c")
    base = wid * QPW

    pltpu.sync_copy(hcx_h, hcx)
    pltpu.sync_copy(hcy_h, hcy)
    pltpu.sync_copy(hcz_h, hcz)
    pltpu.sync_copy(cx_h, cx)
    pltpu.sync_copy(cy_h, cy)
    pltpu.sync_copy(cz_h, cz)
    pltpu.sync_copy(sq2_h, sq2_v)
    pltpu.sync_copy(ids_h, ids_v)
    pltpu.sync_copy(hqx_h.at[pl.ds(base, QPW)], hqx)
    pltpu.sync_copy(hqy_h.at[pl.ds(base, QPW)], hqy)
    pltpu.sync_copy(hqz_h.at[pl.ds(base, QPW)], hqz)
    pltpu.sync_copy(qx_h.at[pl.ds(base, QPW)], qx)
    pltpu.sync_copy(qy_h.at[pl.ds(base, QPW)], qy)
    pltpu.sync_copy(qz_h.at[pl.ds(base, QPW)], qz)
    pltpu.sync_copy(sq1_h.at[pl.ds(base, QPW)], sq1_v)

    iota = lax.iota(jnp.int32, L)
    perms = [jnp.bitwise_and(iota + sh, L - 1) for sh in (8, 4, 2, 1)]
    big_vec = jnp.full((L,), BIG, jnp.float32)
    zero_i = jnp.zeros((L,), jnp.int32)

    def init_body(i, _):
        s = pl.ds(i * L, L)
        colval_v[s] = big_vec
        colidx_v[s] = zero_i
        return 0

    lax.fori_loop(0, N // L, init_body, 0)

    def group_body(g, _):
        qb = g * L
        hqxv = hqx[pl.ds(qb, L)]
        hqyv = hqy[pl.ds(qb, L)]
        hqzv = hqz[pl.ds(qb, L)]
        sq1v = sq1_v[pl.ds(qb, L)]
        res_idx = zero_i
        for sub in range(L // QG):
            lanes = [sub * QG + t for t in range(QG)]
            bqx = [_bcast_lane(hqxv, l) for l in lanes]
            bqy = [_bcast_lane(hqyv, l) for l in lanes]
            bqz = [_bcast_lane(hqzv, l) for l in lanes]
            bs1 = [_bcast_lane(sq1v, l) for l in lanes]
            biv = [jnp.full((L,), qb + l, jnp.int32) for l in lanes]

            def jbody(jb, carry):
                rvals = list(carry[:QG])
                ridxs = list(carry[QG:])
                for u in range(2):
                    s = pl.ds((jb * 2 + u) * L, L)
                    xv = hcx[s]
                    yv = hcy[s]
                    zv = hcz[s]
                    s2 = sq2_v[s]
                    jvec = ids_v[s]
                    cval = colval_v[s]
                    cidx = colidx_v[s]
                    for t in range(QG):
                        m = bs1[t] + s2
                        m = m - bqx[t] * xv
                        m = m - bqy[t] * yv
                        m = m - bqz[t] * zv
                        lt = m < rvals[t]
                        rvals[t] = jnp.where(lt, m, rvals[t])
                        ridxs[t] = jnp.where(lt, jvec, ridxs[t])
                        clt = m < cval
                        cval = jnp.where(clt, m, cval)
                        cidx = jnp.where(clt, biv[t], cidx)
                    colval_v[s] = cval
                    colidx_v[s] = cidx
                return tuple(rvals) + tuple(ridxs)

            carry0 = tuple(big_vec for _ in range(QG)) + \
                tuple(zero_i for _ in range(QG))
            out = lax.fori_loop(0, N // (2 * L), jbody, carry0)
            for t in range(QG):
                _, idx_r = _lane_argmin(out[t], out[QG + t], perms)
                res_idx = jnp.where(iota == lanes[t], idx_r, res_idx)
        gx = plsc.load_gather(cx, [res_idx])
        gy = plsc.load_gather(cy, [res_idx])
        gz = plsc.load_gather(cz, [res_idx])
        dx = qx[pl.ds(qb, L)] - gx
        dy = qy[pl.ds(qb, L)] - gy
        dz = qz[pl.ds(qb, L)] - gz
        rowd2_v[pl.ds(qb, L)] = dx * dx + dy * dy + dz * dz
        return 0

    lax.fori_loop(0, QPW // L, group_body, 0)

    def col_body(jv, _):
        s = pl.ds(jv * L, L)
        lidx = colidx_v[s]
        gx = plsc.load_gather(qx, [lidx])
        gy = plsc.load_gather(qy, [lidx])
        gz = plsc.load_gather(qz, [lidx])
        dx = cx[s] - gx
        dy = cy[s] - gy
        dz = cz[s] - gz
        cold2_v[s] = dx * dx + dy * dy + dz * dz
        return 0

    lax.fori_loop(0, N // L, col_body, 0)

    pltpu.sync_copy(rowd2_v, rowd2_h.at[pl.ds(base, QPW)])
    pltpu.sync_copy(colval_v, colval_h.at[wid])
    pltpu.sync_copy(cold2_v, cold2_h.at[wid])


_phase1 = functools.partial(
    pl.kernel,
    out_type=[
        jax.ShapeDtypeStruct((N,), jnp.float32),
        jax.ShapeDtypeStruct((NW, N), jnp.float32),
        jax.ShapeDtypeStruct((NW, N), jnp.float32),
    ],
    mesh=plsc.VectorSubcoreMesh(core_axis_name="c", subcore_axis_name="s"),
    compiler_params=pltpu.CompilerParams(needs_layout_passes=False),
    scratch_types=[
        pltpu.VMEM((N,), jnp.float32),
        pltpu.VMEM((N,), jnp.float32),
        pltpu.VMEM((N,), jnp.float32),
        pltpu.VMEM((N,), jnp.float32),
        pltpu.VMEM((N,), jnp.float32),
        pltpu.VMEM((N,), jnp.float32),
        pltpu.VMEM((N,), jnp.float32),
        pltpu.VMEM((N,), jnp.int32),
        pltpu.VMEM((N,), jnp.float32),
        pltpu.VMEM((N,), jnp.int32),
        pltpu.VMEM((N,), jnp.float32),
        pltpu.VMEM((QPW,), jnp.float32),
        pltpu.VMEM((QPW,), jnp.float32),
        pltpu.VMEM((QPW,), jnp.float32),
        pltpu.VMEM((QPW,), jnp.float32),
        pltpu.VMEM((QPW,), jnp.float32),
        pltpu.VMEM((QPW,), jnp.float32),
        pltpu.VMEM((QPW,), jnp.float32),
        pltpu.VMEM((QPW,), jnp.float32),
    ],
)(_phase1_body)


_ROWS = N // 128


def _epilogue_body(row_ref, colval_ref, cold2_ref, out_ref):
    val = colval_ref[pl.ds(0, _ROWS), :]
    d2 = cold2_ref[pl.ds(0, _ROWS), :]
    for w in range(1, NW):
        v2 = colval_ref[pl.ds(w * _ROWS, _ROWS), :]
        c2 = cold2_ref[pl.ds(w * _ROWS, _ROWS), :]
        lt = v2 < val
        val = jnp.where(lt, v2, val)
        d2 = jnp.where(lt, c2, d2)
    s1 = jnp.sum(jnp.sqrt(row_ref[...]))
    s2 = jnp.sum(jnp.sqrt(d2))
    out_ref[0, 0] = (s1 + s2) * jnp.float32(1.0 / N)


_epilogue = pl.pallas_call(
    _epilogue_body,
    out_shape=jax.ShapeDtypeStruct((1, 1), jnp.float32),
    in_specs=[
        pl.BlockSpec(memory_space=pltpu.VMEM),
        pl.BlockSpec(memory_space=pltpu.VMEM),
        pl.BlockSpec(memory_space=pltpu.VMEM),
    ],
    out_specs=pl.BlockSpec(memory_space=pltpu.SMEM),
)


def kernel(y1, y2, b1, b2):
    del b1, b2
    h1 = lax.optimization_barrier(y1.astype(jnp.bfloat16)).astype(jnp.float32)
    h2 = lax.optimization_barrier(y2.astype(jnp.bfloat16)).astype(jnp.float32)
    sq1 = jnp.sum(y1 * y1, axis=1)
    sq2 = jnp.sum(y2 * y2, axis=1)
    ids = jnp.arange(N, dtype=jnp.int32)
    h2d = h2 + h2
    rowd2, colval, cold2 = _phase1(
        h1[:, 0], h1[:, 1], h1[:, 2], h2d[:, 0], h2d[:, 1], h2d[:, 2],
        y1[:, 0], y1[:, 1], y1[:, 2], y2[:, 0], y2[:, 1], y2[:, 2],
        sq1, sq2, ids)
    out = _epilogue(rowd2.reshape(_ROWS, 128),
                    colval.reshape(NW * _ROWS, 128),
                    cold2.reshape(NW * _ROWS, 128))
    return out[0, 0]
